# baseline (speedup 1.0000x reference)
import numpy as np
import jax
import jax.numpy as jnp
from jax import lax
from jax.experimental import pallas as pl
from jax.experimental.pallas import tpu as pltpu
from jax.experimental.pallas import tpu_sc as plsc

P = 64
NPTS = P * P
NVAL = NPTS * 3
NPATCH = 96
NW = 32
PPW = NPATCH // NW
CHUNK = 128
NGCH = 4 * NPTS // CHUNK
NSTEP = NPTS // 16
NVSTEP = NVAL // 16
H = 512
W = 512


def _gather_body(img, pf, pi, gout, pfv, piv, gidx, stage, gsem):
    cid = lax.axis_index("c")
    sid = lax.axis_index("s")
    wid = sid * 2 + cid

    lane = lax.iota(jnp.int32, 16)
    half = jnp.float32(1.0 / P)

    def patch_body(t, carry0):
        pid = wid * PPW + t
        pltpu.sync_copy(pf.at[pid], pfv)
        pltpu.sync_copy(pi.at[pid], piv)
        ca = pfv[0, :]
        sa = pfv[1, :]
        sx = pfv[2, :]
        sy = pfv[3, :]
        ext = pfv[4, :]
        px = pfv[5, :]
        py = pfv[6, :]
        bb = piv[0, :]

        def gen(s, carry):
            p = s * 16 + lane
            yi = lax.shift_right_logical(p, 6)
            xi = lax.bitwise_and(p, 63)
            gxv = (xi.astype(jnp.float32) + 0.5) * half - 0.5
            gyv = (yi.astype(jnp.float32) + 0.5) * half - 0.5
            X = gxv * sx
            Y = gyv * sy
            xr = ca * X - sa * Y
            yr = sa * X + ca * Y
            gx2 = px + xr * ext
            gy2 = py + yr * ext
            tx = gx2.astype(jnp.int32)
            txf = tx.astype(jnp.float32)
            ix0 = jnp.where(txf > gx2, tx - 1, tx)
            ty = gy2.astype(jnp.int32)
            tyf = ty.astype(jnp.float32)
            iy0 = jnp.where(tyf > gy2, ty - 1, ty)
            x0 = jnp.clip(ix0, 0, W - 1)
            x1 = jnp.minimum(x0 + 1, W - 1)
            y0 = jnp.clip(iy0, 0, H - 1)
            y1 = jnp.minimum(y0 + 1, H - 1)
            rowa = bb + y0 * W
            rowc = bb + y1 * W
            row = s // 8
            col = (s % 8) * 16
            gidx[row, pl.ds(col, 16)] = rowa + x0
            gidx[32 + row, pl.ds(col, 16)] = rowa + x1
            gidx[64 + row, pl.ds(col, 16)] = rowc + x0
            gidx[96 + row, pl.ds(col, 16)] = rowc + x1
            return carry

        lax.fori_loop(0, NSTEP, gen, 0)

        def group(g, carry):
            def gfire(j, c):
                pltpu.async_copy(img.at[gidx.at[g * 4 + j]],
                                 stage.at[pl.ds(j * CHUNK, CHUNK)], gsem)
                return c

            lax.fori_loop(0, 4, gfire, 0)

            def gdrain(j, c):
                pltpu.make_async_copy(img.at[gidx.at[0]],
                                      stage.at[pl.ds(0, CHUNK)], gsem).wait()
                return c

            lax.fori_loop(0, 4, gdrain, 0)
            pltpu.sync_copy(
                stage,
                gout.at[pl.ds(pid * (4 * NPTS) + g * (4 * CHUNK), 4 * CHUNK)])
            return carry

        lax.fori_loop(0, NGCH // 4, group, 0)
        return carry0

    lax.fori_loop(0, PPW, patch_body, 0)


def _blend_body(gv_hbm, pf, pi, oout, pfv, piv, gvb, vals, red):
    cid = lax.axis_index("c")
    sid = lax.axis_index("s")
    wid = sid * 2 + cid

    lane = lax.iota(jnp.int32, 16)
    half = jnp.float32(1.0 / P)
    NP8 = NPTS * 8
    CH = NP8 // 4
    NSCH8 = CH // 16

    def patch_body(t, carry0):
        pid = wid * PPW + t
        pltpu.sync_copy(pf.at[pid], pfv)
        ca = pfv[0, :]
        sa = pfv[1, :]
        sx = pfv[2, :]
        sy = pfv[3, :]
        ext = pfv[4, :]
        px = pfv[5, :]
        py = pfv[6, :]
        pbase = pid * (4 * NP8)

        def chunk_body(chv, carryc):
            smc, sqc = carryc

            def load_plane(q, c):
                pltpu.sync_copy(
                    gv_hbm.at[pl.ds(pbase + q * NP8 + chv * CH, CH)],
                    gvb.at[pl.ds(q * CH, CH)])
                return c

            lax.fori_loop(0, 4, load_plane, 0)

            def blend(s, carry):
                sm, sq = carry
                u16 = chv * CH + s * 16 + lane
                p = lax.shift_right_logical(u16, 3)
                yi = lax.shift_right_logical(p, 6)
                xi = lax.bitwise_and(p, 63)
                gxv = (xi.astype(jnp.float32) + 0.5) * half - 0.5
                gyv = (yi.astype(jnp.float32) + 0.5) * half - 0.5
                X = gxv * sx
                Y = gyv * sy
                xr = ca * X - sa * Y
                yr = sa * X + ca * Y
                gx2 = px + xr * ext
                gy2 = py + yr * ext
                tx = gx2.astype(jnp.int32)
                txf = tx.astype(jnp.float32)
                fx = jnp.where(txf > gx2, txf - 1.0, txf)
                ty = gy2.astype(jnp.int32)
                tyf = ty.astype(jnp.float32)
                fy = jnp.where(tyf > gy2, tyf - 1.0, tyf)
                wx = gx2 - fx
                wy = gy2 - fy
                off = s * 16
                ia = gvb[pl.ds(off, 16)]
                ib = gvb[pl.ds(CH + off, 16)]
                ic = gvb[pl.ds(2 * CH + off, 16)]
                idd = gvb[pl.ds(3 * CH + off, 16)]
                top = ia + wx * (ib - ia)
                bot = ic + wx * (idd - ic)
                val = top + wy * (bot - top)
                vals[pl.ds(chv * CH + off, 16)] = val
                return (sm + val, sq + val * val)

            return lax.fori_loop(0, NSCH8, blend, (smc, sqc))

        zero16 = lane.astype(jnp.float32) * 0.0
        sm, sq = lax.fori_loop(0, 4, chunk_body, (zero16, zero16))

        def allsum(v):
            for sh in (8, 4, 2, 1):
                red[pl.ds(0, 16)] = v
                red[pl.ds(16, 16)] = v
                v = v + red[pl.ds(sh, 16)]
            return v

        inv_n = jnp.float32(1.0 / NVAL)
        meanv = allsum(sm) * inv_n
        varv = jnp.maximum(allsum(sq) * inv_n - meanv * meanv, 0.0)
        y = zero16 + 2.0
        for _ in range(7):
            y = jnp.where(varv * y * y < 0.22, y * 2.0, y)
        for _ in range(6):
            y = y * (1.5 - 0.5 * varv * y * y)
        factor = jnp.minimum(y, jnp.float32(float(np.sqrt(NVAL))))

        def norm(s, carry):
            v = vals[pl.ds(s * 16, 16)]
            vals[pl.ds(s * 16, 16)] = (v - meanv) * factor
            return carry

        lax.fori_loop(0, NP8 // 16, norm, 0)

        pltpu.sync_copy(vals, oout.at[pid])
        return carry0

    lax.fori_loop(0, PPW, patch_body, 0)


def kernel(image_input, scale, hexshape, pos, hexrot):
    B = image_input.shape[0]
    scale = jnp.clip(scale, 1.0, 4.0)
    pos = jnp.clip(pos, 0.0, 1.0)

    k3 = jnp.arange(3, dtype=jnp.float32)
    ang = hexrot[:, 0:1] + k3[None, :] * jnp.float32(2.0 * np.pi / 3.0)
    ca = jnp.cos(ang)
    sa = jnp.sin(ang)

    def rep(v):
        return jnp.broadcast_to(v[:, None], (B, 3))

    sx = rep(0.5 + hexshape[:, 0])
    sy = rep(0.5 + hexshape[:, 1])
    ext = rep(scale[:, 0] * np.float32(P))
    px = rep(pos[:, 0] * np.float32(W - 1))
    py = rep(pos[:, 1] * np.float32(H - 1))
    pad = jnp.zeros((B, 3), jnp.float32)

    pf = jnp.stack([ca, sa, sx, sy, ext, px, py, pad], axis=2)
    pf = jnp.broadcast_to(pf.reshape(B * 3, 8, 1), (B * 3, 8, 16))
    pf = pf.astype(jnp.float32)

    b = jnp.arange(B, dtype=jnp.int32)
    bb = jnp.broadcast_to((b * (H * W))[:, None], (B, 3))
    pi = jnp.broadcast_to(bb.reshape(B * 3, 1, 1), (B * 3, 1, 16))
    pi = pi.astype(jnp.int32)

    img_flat = image_input.reshape(B * H * W, 3)
    emb = jnp.asarray(np.eye(3, 8, dtype=np.float32))
    img8 = jax.numpy.matmul(img_flat, emb)

    mesh = plsc.VectorSubcoreMesh(core_axis_name="c", subcore_axis_name="s")

    gather = pl.kernel(
        _gather_body,
        out_type=jax.ShapeDtypeStruct((NPATCH * 4 * NPTS, 8), jnp.float32),
        mesh=mesh,
        compiler_params=pltpu.CompilerParams(use_tc_tiling_on_sc=False),
        scratch_types=[
            pltpu.VMEM((8, 16), jnp.float32),
            pltpu.VMEM((1, 16), jnp.int32),
            pltpu.VMEM((NGCH, CHUNK), jnp.int32),
            pltpu.VMEM((4 * CHUNK, 8), jnp.float32),
            pltpu.SemaphoreType.DMA,
        ],
    )
    gout = gather(img8, pf, pi)

    e3 = np.zeros((16, 8, 128), np.float32)
    for _r in range(16):
        for _k in range(8):
            e3[_r, _k, _r * 8 + _k] = 1.0
    g3 = gout.reshape(NPATCH * 4 * NPTS // 16, 16, 8)
    gflat2d = jnp.einsum('mrk,rku->mu', g3, jnp.asarray(e3),
                         precision=lax.Precision.HIGHEST)
    gflat = gflat2d.reshape(NPATCH * 4 * NPTS * 8)

    blend = pl.kernel(
        _blend_body,
        out_type=jax.ShapeDtypeStruct((NPATCH, NPTS * 8), jnp.float32),
        mesh=mesh,
        compiler_params=pltpu.CompilerParams(use_tc_tiling_on_sc=False),
        scratch_types=[
            pltpu.VMEM((8, 16), jnp.float32),
            pltpu.VMEM((1, 16), jnp.int32),
            pltpu.VMEM((4 * (NPTS * 8) // 4,), jnp.float32),
            pltpu.VMEM((NPTS * 8,), jnp.float32),
            pltpu.VMEM((32,), jnp.float32),
        ],
    )
    oout = blend(gflat, pf, pi)

    emb_t = jnp.asarray(np.eye(8, 3, dtype=np.float32))
    o4 = oout.reshape(B, 3, P * P, 8)
    out = jnp.einsum('bkpj,jc->bpkc', o4, emb_t,
                     precision=lax.Precision.HIGHEST)
    return out.reshape(-1, P, P, 3)


# speedup vs baseline: 1.1551x; 1.1551x over previous
import numpy as np
import jax
import jax.numpy as jnp
from jax import lax
from jax.experimental import pallas as pl
from jax.experimental.pallas import tpu as pltpu
from jax.experimental.pallas import tpu_sc as plsc

P = 64
NPTS = P * P
NVAL = NPTS * 3
NPATCH = 96
NW = 32
PPW = NPATCH // NW
CHUNK = 128
NGCH = 4 * NPTS // CHUNK
NSTEP = NPTS // 16
NVSTEP = NVAL // 16
H = 512
W = 512


def _gather_body(img, pf, pi, gout, pfv, piv, gidx, stage, gsem):
    cid = lax.axis_index("c")
    sid = lax.axis_index("s")
    wid = sid * 2 + cid

    lane = lax.iota(jnp.int32, 16)
    half = jnp.float32(1.0 / P)

    def patch_body(t, carry0):
        pid = wid * PPW + t
        pltpu.sync_copy(pf.at[pid], pfv)
        pltpu.sync_copy(pi.at[pid], piv)
        ca = pfv[0, :]
        sa = pfv[1, :]
        sx = pfv[2, :]
        sy = pfv[3, :]
        ext = pfv[4, :]
        px = pfv[5, :]
        py = pfv[6, :]
        bb = piv[0, :]

        def gen(s, carry):
            p = s * 16 + lane
            yi = lax.shift_right_logical(p, 6)
            xi = lax.bitwise_and(p, 63)
            gxv = (xi.astype(jnp.float32) + 0.5) * half - 0.5
            gyv = (yi.astype(jnp.float32) + 0.5) * half - 0.5
            X = gxv * sx
            Y = gyv * sy
            xr = ca * X - sa * Y
            yr = sa * X + ca * Y
            gx2 = px + xr * ext
            gy2 = py + yr * ext
            tx = gx2.astype(jnp.int32)
            txf = tx.astype(jnp.float32)
            ix0 = jnp.where(txf > gx2, tx - 1, tx)
            ty = gy2.astype(jnp.int32)
            tyf = ty.astype(jnp.float32)
            iy0 = jnp.where(tyf > gy2, ty - 1, ty)
            x0 = jnp.clip(ix0, 0, W - 1)
            x1 = jnp.minimum(x0 + 1, W - 1)
            y0 = jnp.clip(iy0, 0, H - 1)
            y1 = jnp.minimum(y0 + 1, H - 1)
            rowa = bb + y0 * W
            rowc = bb + y1 * W
            row = s // 8
            col = (s % 8) * 16
            gidx[row, pl.ds(col, 16)] = rowa + x0
            gidx[32 + row, pl.ds(col, 16)] = rowa + x1
            gidx[64 + row, pl.ds(col, 16)] = rowc + x0
            gidx[96 + row, pl.ds(col, 16)] = rowc + x1
            return carry

        lax.fori_loop(0, NSTEP, gen, 0)

        def group(g, carry):
            def gfire(j, c):
                pltpu.async_copy(img.at[gidx.at[g * 4 + j]],
                                 stage.at[pl.ds(j * CHUNK, CHUNK)], gsem)
                return c

            lax.fori_loop(0, 4, gfire, 0)

            def gdrain(j, c):
                pltpu.make_async_copy(img.at[gidx.at[0]],
                                      stage.at[pl.ds(0, CHUNK)], gsem).wait()
                return c

            lax.fori_loop(0, 4, gdrain, 0)
            pltpu.sync_copy(
                stage,
                gout.at[pl.ds(pid * (4 * NPTS) + g * (4 * CHUNK), 4 * CHUNK)])
            return carry

        lax.fori_loop(0, NGCH // 4, group, 0)
        return carry0

    lax.fori_loop(0, PPW, patch_body, 0)


def _blend_body(gv_hbm, pf, pi, oout, pfv, piv, gvb, vals, red):
    cid = lax.axis_index("c")
    sid = lax.axis_index("s")
    wid = sid * 2 + cid

    lane = lax.iota(jnp.int32, 16)
    half = jnp.float32(1.0 / P)
    NP8 = NPTS * 8
    CH = NP8 // 4
    NSCH8 = CH // 16

    def patch_body(t, carry0):
        pid = wid * PPW + t
        pltpu.sync_copy(pf.at[pid], pfv)
        ca = pfv[0, :]
        sa = pfv[1, :]
        sx = pfv[2, :]
        sy = pfv[3, :]
        ext = pfv[4, :]
        px = pfv[5, :]
        py = pfv[6, :]
        pbase = pid * (4 * NP8)

        def chunk_body(chv, carryc):
            smc, sqc = carryc

            def load_plane(q, c):
                pltpu.sync_copy(
                    gv_hbm.at[pl.ds(pbase + q * NP8 + chv * CH, CH)],
                    gvb.at[pl.ds(q * CH, CH)])
                return c

            lax.fori_loop(0, 4, load_plane, 0)

            def blend(s, carry):
                sm, sq = carry
                u16 = chv * CH + s * 16 + lane
                p = lax.shift_right_logical(u16, 3)
                yi = lax.shift_right_logical(p, 6)
                xi = lax.bitwise_and(p, 63)
                gxv = (xi.astype(jnp.float32) + 0.5) * half - 0.5
                gyv = (yi.astype(jnp.float32) + 0.5) * half - 0.5
                X = gxv * sx
                Y = gyv * sy
                xr = ca * X - sa * Y
                yr = sa * X + ca * Y
                gx2 = px + xr * ext
                gy2 = py + yr * ext
                tx = gx2.astype(jnp.int32)
                txf = tx.astype(jnp.float32)
                fx = jnp.where(txf > gx2, txf - 1.0, txf)
                ty = gy2.astype(jnp.int32)
                tyf = ty.astype(jnp.float32)
                fy = jnp.where(tyf > gy2, tyf - 1.0, tyf)
                wx = gx2 - fx
                wy = gy2 - fy
                off = s * 16
                ia = gvb[pl.ds(off, 16)]
                ib = gvb[pl.ds(CH + off, 16)]
                ic = gvb[pl.ds(2 * CH + off, 16)]
                idd = gvb[pl.ds(3 * CH + off, 16)]
                top = ia + wx * (ib - ia)
                bot = ic + wx * (idd - ic)
                val = top + wy * (bot - top)
                vals[pl.ds(chv * CH + off, 16)] = val
                return (sm + val, sq + val * val)

            return lax.fori_loop(0, NSCH8, blend, (smc, sqc))

        zero16 = lane.astype(jnp.float32) * 0.0
        sm, sq = lax.fori_loop(0, 4, chunk_body, (zero16, zero16))

        def allsum(v):
            for sh in (8, 4, 2, 1):
                red[pl.ds(0, 16)] = v
                red[pl.ds(16, 16)] = v
                v = v + red[pl.ds(sh, 16)]
            return v

        inv_n = jnp.float32(1.0 / NVAL)
        meanv = allsum(sm) * inv_n
        varv = jnp.maximum(allsum(sq) * inv_n - meanv * meanv, 0.0)
        y = zero16 + 2.0
        for _ in range(7):
            y = jnp.where(varv * y * y < 0.22, y * 2.0, y)
        for _ in range(6):
            y = y * (1.5 - 0.5 * varv * y * y)
        factor = jnp.minimum(y, jnp.float32(float(np.sqrt(NVAL))))

        def norm(s, carry):
            v = vals[pl.ds(s * 16, 16)]
            vals[pl.ds(s * 16, 16)] = (v - meanv) * factor
            return carry

        lax.fori_loop(0, NP8 // 16, norm, 0)

        pltpu.sync_copy(vals, oout.at[pid])
        return carry0

    lax.fori_loop(0, PPW, patch_body, 0)


def kernel(image_input, scale, hexshape, pos, hexrot):
    B = image_input.shape[0]
    scale = jnp.clip(scale, 1.0, 4.0)
    pos = jnp.clip(pos, 0.0, 1.0)

    k3 = jnp.arange(3, dtype=jnp.float32)
    ang = hexrot[:, 0:1] + k3[None, :] * jnp.float32(2.0 * np.pi / 3.0)
    ca = jnp.cos(ang)
    sa = jnp.sin(ang)

    def rep(v):
        return jnp.broadcast_to(v[:, None], (B, 3))

    sx = rep(0.5 + hexshape[:, 0])
    sy = rep(0.5 + hexshape[:, 1])
    ext = rep(scale[:, 0] * np.float32(P))
    px = rep(pos[:, 0] * np.float32(W - 1))
    py = rep(pos[:, 1] * np.float32(H - 1))
    pad = jnp.zeros((B, 3), jnp.float32)

    pf = jnp.stack([ca, sa, sx, sy, ext, px, py, pad], axis=2)
    pf = jnp.broadcast_to(pf.reshape(B * 3, 8, 1), (B * 3, 8, 16))
    pf = pf.astype(jnp.float32)

    b = jnp.arange(B, dtype=jnp.int32)
    bb = jnp.broadcast_to((b * (H * W))[:, None], (B, 3))
    pi = jnp.broadcast_to(bb.reshape(B * 3, 1, 1), (B * 3, 1, 16))
    pi = pi.astype(jnp.int32)

    img_flat = image_input.reshape(B * H * W, 3)
    emb = jnp.asarray(np.eye(3, 8, dtype=np.float32))
    img8 = jax.numpy.matmul(img_flat, emb)

    mesh = plsc.VectorSubcoreMesh(core_axis_name="c", subcore_axis_name="s")

    gather = pl.kernel(
        _gather_body,
        out_type=jax.ShapeDtypeStruct((NPATCH * 4 * NPTS, 8), jnp.float32),
        mesh=mesh,
        compiler_params=pltpu.CompilerParams(use_tc_tiling_on_sc=False),
        scratch_types=[
            pltpu.VMEM((8, 16), jnp.float32),
            pltpu.VMEM((1, 16), jnp.int32),
            pltpu.VMEM((NGCH, CHUNK), jnp.int32),
            pltpu.VMEM((4 * CHUNK, 8), jnp.float32),
            pltpu.SemaphoreType.DMA,
        ],
    )
    gout = gather(img8, pf, pi)

    gflat = gout.reshape(NPATCH * 4 * NPTS * 8)

    blend = pl.kernel(
        _blend_body,
        out_type=jax.ShapeDtypeStruct((NPATCH, NPTS * 8), jnp.float32),
        mesh=mesh,
        compiler_params=pltpu.CompilerParams(use_tc_tiling_on_sc=False),
        scratch_types=[
            pltpu.VMEM((8, 16), jnp.float32),
            pltpu.VMEM((1, 16), jnp.int32),
            pltpu.VMEM((4 * (NPTS * 8) // 4,), jnp.float32),
            pltpu.VMEM((NPTS * 8,), jnp.float32),
            pltpu.VMEM((32,), jnp.float32),
        ],
    )
    oout = blend(gflat, pf, pi)

    emb_t = jnp.asarray(np.eye(8, 3, dtype=np.float32))
    o4 = oout.reshape(B, 3, P * P, 8)
    out = jnp.einsum('bkpj,jc->bpkc', o4, emb_t,
                     precision=lax.Precision.HIGHEST)
    return out.reshape(-1, P, P, 3)
